# baseline (speedup 1.0000x reference)
import functools

import jax
import jax.numpy as jnp
from jax import lax
from jax.experimental import pallas as pl
from jax.experimental.pallas import tpu as pltpu
from jax.experimental.pallas import tpu_sc as plsc

VOCAB = 1000000
DIM = 32
B = 16384
CTX = 10
NEG = 5

NC = 2
NS = 16
L = 16
NW = NC * NS
BPW = B // NW
C = 128
NCHUNK = BPW // C
G = C // L


def _log_sigmoid(x):
    u = jnp.exp(-jnp.abs(x))
    z = u / (u + 2.0)
    z2 = z * z
    at = z * (1.0 + z2 * (1.0 / 3.0 + z2 * (0.2 + z2 * (1.0 / 7.0 + z2 * (1.0 / 9.0)))))
    return jnp.minimum(x, 0.0) - 2.0 * at


def _body(ctx_idx_h, tgt_idx_h, neg_idx_h, ctx_tab_h, tgt_tab_h, out_h,
          ctx_idx_v, tgt_idx_v, neg_idx_v, ctx_rows, tgt_rows, neg_rows,
          loss_v, sem):
    wid = lax.axis_index("s") * NC + lax.axis_index("c")
    lane = lax.broadcasted_iota(jnp.int32, (L,), 0)

    def chunk_body(ci, carry):
        base = wid * BPW + ci * C
        pltpu.sync_copy(ctx_idx_h.at[:, pl.ds(base, C)], ctx_idx_v)
        pltpu.sync_copy(neg_idx_h.at[:, pl.ds(base, C)], neg_idx_v)
        pltpu.sync_copy(tgt_idx_h.at[pl.ds(base, C)], tgt_idx_v)
        cps = []
        for j in range(CTX):
            cps.append(pltpu.async_copy(ctx_tab_h.at[ctx_idx_v.at[j]],
                                        ctx_rows.at[j], sem))
        cps.append(pltpu.async_copy(tgt_tab_h.at[tgt_idx_v], tgt_rows, sem))
        for n in range(NEG):
            cps.append(pltpu.async_copy(tgt_tab_h.at[neg_idx_v.at[n]],
                                        neg_rows.at[n], sem))
        for cp in cps:
            cp.wait()

        def group_body(g, gcarry):
            e0 = g * L
            pos_v = jnp.zeros((L,), jnp.float32)
            negs_v = [jnp.zeros((L,), jnp.float32) for _ in range(NEG)]
            for e16 in range(L):
                e = e0 + e16
                h_lo = ctx_rows[0, e, pl.ds(0, L)]
                h_hi = ctx_rows[0, e, pl.ds(L, L)]
                for j in range(1, CTX):
                    h_lo = h_lo + ctx_rows[j, e, pl.ds(0, L)]
                    h_hi = h_hi + ctx_rows[j, e, pl.ds(L, L)]
                t_lo = tgt_rows[e, pl.ds(0, L)]
                t_hi = tgt_rows[e, pl.ds(L, L)]
                ps = jnp.sum(h_lo * t_lo + h_hi * t_hi)
                pos_v = jnp.where(lane == e16, ps, pos_v)
                for n in range(NEG):
                    n_lo = neg_rows[n, e, pl.ds(0, L)]
                    n_hi = neg_rows[n, e, pl.ds(L, L)]
                    ns = jnp.sum(h_lo * n_lo + h_hi * n_hi)
                    negs_v[n] = jnp.where(lane == e16, ns, negs_v[n])
            scale = 1.0 / CTX
            acc = _log_sigmoid(pos_v * scale)
            for n in range(NEG):
                acc = acc + _log_sigmoid(-(negs_v[n] * scale))
            loss_v[pl.ds(e0, L)] = -acc
            return gcarry

        lax.fori_loop(0, G, group_body, 0, unroll=False)
        pltpu.sync_copy(loss_v, out_h.at[pl.ds(base, C)])
        return carry

    lax.fori_loop(0, NCHUNK, chunk_body, 0, unroll=False)


_sc_call = pl.kernel(
    _body,
    out_type=jax.ShapeDtypeStruct((B,), jnp.float32),
    mesh=plsc.VectorSubcoreMesh(core_axis_name="c", subcore_axis_name="s",
                                num_cores=NC, num_subcores=NS),
    scratch_types=[
        pltpu.VMEM((CTX, C), jnp.int32),
        pltpu.VMEM((C,), jnp.int32),
        pltpu.VMEM((NEG, C), jnp.int32),
        pltpu.VMEM((CTX, C, DIM), jnp.float32),
        pltpu.VMEM((C, DIM), jnp.float32),
        pltpu.VMEM((NEG, C, DIM), jnp.float32),
        pltpu.VMEM((C,), jnp.float32),
        pltpu.SemaphoreType.DMA,
    ],
    compiler_params=pltpu.CompilerParams(needs_layout_passes=False,
                                         use_tc_tiling_on_sc=False),
)


def kernel(context_idx, target_idx, neg_idx, context_vectors, target_vectors):
    ctx_t = jnp.asarray(context_idx, jnp.int32).T
    neg_t = jnp.asarray(neg_idx, jnp.int32).T
    tgt = jnp.asarray(target_idx, jnp.int32)
    return _sc_call(ctx_t, tgt, neg_t, context_vectors, target_vectors)
